# baseline (speedup 1.0000x reference)
import functools

import jax
import jax.numpy as jnp
from jax import lax
from jax.experimental import pallas as pl
from jax.experimental.pallas import tpu as pltpu
from jax.experimental.pallas import tpu_sc as plsc

_NS = 16
_L = 16


def _pg_body(rows_per_w, vocab, lp_hbm, tgt_hbm, rwd_hbm, out_hbm,
             tgt_v, rwd_v, idx_v, val_v, acc_v, sem_t, sem_r, sem_g):
    wid = lax.axis_index("s")
    base = wid * rows_per_w

    cp_t = pltpu.make_async_copy(tgt_hbm.at[pl.ds(base, rows_per_w)], tgt_v, sem_t)
    cp_r = pltpu.make_async_copy(rwd_hbm.at[pl.ds(base, rows_per_w)], rwd_v, sem_r)
    cp_t.start()
    cp_r.start()
    cp_t.wait()

    nvec = rows_per_w // _L
    ct = vocab // 128
    for j in range(nvec):
        t = tgt_v[pl.ds(j * _L, _L)]
        rows = (base + j * _L) + lax.iota(jnp.int32, _L)
        idx_v[pl.ds(j * _L, _L)] = (((rows >> 3) * ct + (t >> 7)) * 1024
                                    + ((rows & 7) << 7) + (t & 127))

    pltpu.async_copy(lp_hbm.at[idx_v], val_v, sem_g).wait()
    cp_r.wait()

    acc = val_v[pl.ds(0, _L)] * rwd_v[pl.ds(0, _L)]
    for j in range(1, nvec):
        acc = acc + val_v[pl.ds(j * _L, _L)] * rwd_v[pl.ds(j * _L, _L)]
    acc_v[...] = acc
    pltpu.sync_copy(acc_v, out_hbm.at[wid])


def kernel(log_pred, target, reward, seq_len):
    n_rows, vocab = log_pred.shape
    rows_per_w = n_rows // _NS

    lp_flat = (log_pred
               .reshape(n_rows // 8, 8, vocab // 128, 128)
               .transpose(0, 2, 1, 3)
               .reshape(-1))
    tgt_flat = target.reshape(-1).astype(jnp.int32)
    scale = -1.0 / (seq_len * target.shape[0]).astype(jnp.float32)
    rwd_flat = reward.reshape(-1) * scale

    mesh = plsc.VectorSubcoreMesh(
        core_axis_name="c", subcore_axis_name="s", num_cores=1)
    body = functools.partial(_pg_body, rows_per_w, vocab)
    partials = pl.kernel(
        body,
        out_type=jax.ShapeDtypeStruct((_NS, _L), jnp.float32),
        mesh=mesh,
        scratch_types=[
            pltpu.VMEM((rows_per_w,), jnp.int32),
            pltpu.VMEM((rows_per_w,), jnp.float32),
            pltpu.VMEM((rows_per_w,), jnp.int32),
            pltpu.VMEM((rows_per_w,), jnp.float32),
            pltpu.VMEM((_L,), jnp.float32),
            pltpu.SemaphoreType.DMA,
            pltpu.SemaphoreType.DMA,
            pltpu.SemaphoreType.DMA,
        ],
    )(lp_flat, tgt_flat, rwd_flat)

    return jnp.sum(partials)


# speedup vs baseline: 1.0180x; 1.0180x over previous
import functools

import jax
import jax.numpy as jnp
from jax import lax
from jax.experimental import pallas as pl
from jax.experimental.pallas import tpu as pltpu
from jax.experimental.pallas import tpu_sc as plsc

_NS = 16
_L = 16


def _pg_body(rows_per_w, lp_hbm, packed_i_hbm, packed_f_hbm, out_hbm,
             idx_v, rwd_v, val_v, acc_v, sem_a, sem_r, sem_g):
    wid = lax.axis_index("s")
    blk = 2 * rows_per_w

    cp_a = pltpu.make_async_copy(
        packed_i_hbm.at[pl.ds(wid * blk, rows_per_w)], idx_v, sem_a)
    cp_r = pltpu.make_async_copy(
        packed_f_hbm.at[pl.ds(wid * blk + rows_per_w, rows_per_w)], rwd_v, sem_r)
    cp_a.start()
    cp_r.start()
    cp_a.wait()

    pltpu.async_copy(lp_hbm.at[idx_v], val_v, sem_g).wait()
    cp_r.wait()

    nvec = rows_per_w // _L
    acc = val_v[pl.ds(0, _L)] * rwd_v[pl.ds(0, _L)]
    for j in range(1, nvec):
        acc = acc + val_v[pl.ds(j * _L, _L)] * rwd_v[pl.ds(j * _L, _L)]
    acc_v[...] = acc
    pltpu.sync_copy(acc_v, out_hbm.at[wid])


def kernel(log_pred, target, reward, seq_len):
    n_rows, vocab = log_pred.shape
    rows_per_w = n_rows // _NS

    lp_flat = (log_pred
               .reshape(n_rows // 8, 8, vocab // 128, 128)
               .transpose(0, 2, 1, 3)
               .reshape(-1))

    t = target.reshape(-1).astype(jnp.int32)
    rows = lax.iota(jnp.int32, n_rows)
    ct = vocab // 128
    addr = (((rows >> 3) * ct + (t >> 7)) * 1024
            + ((rows & 7) << 7) + (t & 127))
    scale = -1.0 / (seq_len * target.shape[0]).astype(jnp.float32)
    rwd_bits = lax.bitcast_convert_type(reward.reshape(-1) * scale, jnp.int32)
    packed = jnp.concatenate(
        [addr.reshape(_NS, 1, rows_per_w),
         rwd_bits.reshape(_NS, 1, rows_per_w)], axis=1).reshape(-1)

    mesh = plsc.VectorSubcoreMesh(
        core_axis_name="c", subcore_axis_name="s", num_cores=1)
    body = functools.partial(_pg_body, rows_per_w)
    partials = pl.kernel(
        body,
        out_type=jax.ShapeDtypeStruct((_NS, _L), jnp.float32),
        mesh=mesh,
        scratch_types=[
            pltpu.VMEM((rows_per_w,), jnp.int32),
            pltpu.VMEM((rows_per_w,), jnp.float32),
            pltpu.VMEM((rows_per_w,), jnp.float32),
            pltpu.VMEM((_L,), jnp.float32),
            pltpu.SemaphoreType.DMA,
            pltpu.SemaphoreType.DMA,
            pltpu.SemaphoreType.DMA,
        ],
    )(lp_flat, packed, lax.bitcast_convert_type(packed, jnp.float32))

    return jnp.sum(partials)
